# baseline (speedup 1.0000x reference)
import functools

import jax
import jax.numpy as jnp
from jax import lax
from jax.experimental import pallas as pl
from jax.experimental.pallas import tpu as pltpu
from jax.experimental.pallas import tpu_sc as plsc

SIZE = 32
LANES = 16
NUM_CORES = 2
NUM_SUBCORES = 16
NUM_WORKERS = NUM_CORES * NUM_SUBCORES
IDX_CHUNK = 128


@functools.partial(jax.jit, static_argnums=0)
def _mf_sc(batch, uid, iid, uebd_t, iebd_t, ubias, ibias, mean16):
    b_per_w = batch // NUM_WORKERS
    n_chunks = b_per_w // IDX_CHUNK
    n_groups = b_per_w // LANES
    mesh = plsc.VectorSubcoreMesh(core_axis_name="c", subcore_axis_name="s")

    @functools.partial(
        pl.kernel,
        mesh=mesh,
        compiler_params=pltpu.CompilerParams(
            needs_layout_passes=False, use_tc_tiling_on_sc=False),
        out_type=jax.ShapeDtypeStruct((batch,), jnp.float32),
        scratch_types=[
            pltpu.VMEM((n_chunks, IDX_CHUNK), jnp.int32),
            pltpu.VMEM((n_chunks, IDX_CHUNK), jnp.int32),
            pltpu.VMEM((SIZE, IDX_CHUNK), jnp.float32),
            pltpu.VMEM((SIZE, IDX_CHUNK), jnp.float32),
            pltpu.VMEM((b_per_w,), jnp.float32),
            pltpu.VMEM((b_per_w,), jnp.float32),
            pltpu.VMEM((LANES,), jnp.float32),
            pltpu.VMEM((b_per_w,), jnp.float32),
            pltpu.SemaphoreType.DMA,
            pltpu.SemaphoreType.DMA,
        ],
    )
    def mf(uid_hbm, iid_hbm, uebd_hbm, iebd_hbm, ubias_hbm, ibias_hbm,
           mean_hbm, out_hbm, uidx, iidx, urows, irows, ubv, ibv, meanv,
           outv, sem, bsem):
        wid = lax.axis_index("s") * NUM_CORES + lax.axis_index("c")
        pltpu.sync_copy(uid_hbm.at[wid], uidx)
        pltpu.sync_copy(iid_hbm.at[wid], iidx)
        pltpu.sync_copy(mean_hbm, meanv)
        bias_copies = []
        for c in range(n_chunks):
            sl = pl.ds(c * IDX_CHUNK, IDX_CHUNK)
            bias_copies.append(
                pltpu.async_copy(ubias_hbm.at[uidx.at[c]], ubv.at[sl], bsem))
            bias_copies.append(
                pltpu.async_copy(ibias_hbm.at[iidx.at[c]], ibv.at[sl], bsem))
        for cp in bias_copies:
            cp.wait()
        mean_vec = meanv[...]

        for c in range(n_chunks):
            copies = []
            for j in range(SIZE):
                copies.append(pltpu.async_copy(
                    uebd_hbm.at[j].at[uidx.at[c]], urows.at[j], sem))
                copies.append(pltpu.async_copy(
                    iebd_hbm.at[j].at[iidx.at[c]], irows.at[j], sem))
            for cp in copies:
                cp.wait()

            def cgroup(g, carry):
                sl = pl.ds(g * LANES, LANES)
                acc = jnp.zeros((LANES,), jnp.float32)
                for j in range(SIZE):
                    acc = acc + urows[j, sl] * irows[j, sl]
                obase = pl.ds(c * IDX_CHUNK + g * LANES, LANES)
                outv[obase] = acc + ubv[obase] + ibv[obase] + mean_vec
                return carry

            lax.fori_loop(0, IDX_CHUNK // LANES, cgroup, 0)

        pltpu.sync_copy(outv, out_hbm.at[pl.ds(wid * b_per_w, b_per_w)])

    return mf(uid, iid, uebd_t, iebd_t, ubias, ibias, mean16)


def kernel(user_id, item_id, user_ebds, item_ebds, user_bias, item_bias, mean):
    batch = user_id.shape[0]
    b_per_w = batch // NUM_WORKERS
    uid = user_id.astype(jnp.int32).reshape(
        NUM_WORKERS, b_per_w // IDX_CHUNK, IDX_CHUNK)
    iid = item_id.astype(jnp.int32).reshape(
        NUM_WORKERS, b_per_w // IDX_CHUNK, IDX_CHUNK)
    mean16 = jnp.broadcast_to(mean.astype(jnp.float32), (LANES,))
    return _mf_sc(batch, uid, iid, user_ebds.T, item_ebds.T,
                  user_bias.reshape(-1), item_bias.reshape(-1), mean16)


# speedup vs baseline: 1.0015x; 1.0015x over previous
import functools

import jax
import jax.numpy as jnp
from jax import lax
from jax.experimental import pallas as pl
from jax.experimental.pallas import tpu as pltpu
from jax.experimental.pallas import tpu_sc as plsc

SIZE = 32
LANES = 16
NUM_CORES = 2
NUM_SUBCORES = 16
NUM_WORKERS = NUM_CORES * NUM_SUBCORES
IDX_CHUNK = 128


@functools.partial(jax.jit, static_argnums=0)
def _mf_sc(batch, uid, iid, uebd_t, iebd_t, ubias, ibias, mean16):
    b_per_w = batch // NUM_WORKERS
    n_chunks = b_per_w // IDX_CHUNK
    n_groups = b_per_w // LANES
    mesh = plsc.VectorSubcoreMesh(core_axis_name="c", subcore_axis_name="s")

    @functools.partial(
        pl.kernel,
        mesh=mesh,
        compiler_params=pltpu.CompilerParams(
            needs_layout_passes=False, use_tc_tiling_on_sc=False),
        out_type=jax.ShapeDtypeStruct((batch,), jnp.float32),
        scratch_types=[
            pltpu.VMEM((n_chunks, IDX_CHUNK), jnp.int32),
            pltpu.VMEM((n_chunks, IDX_CHUNK), jnp.int32),
            pltpu.VMEM((SIZE, IDX_CHUNK), jnp.float32),
            pltpu.VMEM((SIZE, IDX_CHUNK), jnp.float32),
            pltpu.VMEM((b_per_w,), jnp.float32),
            pltpu.VMEM((b_per_w,), jnp.float32),
            pltpu.VMEM((LANES,), jnp.float32),
            pltpu.VMEM((b_per_w,), jnp.float32),
            pltpu.SemaphoreType.DMA,
            pltpu.SemaphoreType.DMA,
        ],
    )
    def mf(uid_hbm, iid_hbm, uebd_hbm, iebd_hbm, ubias_hbm, ibias_hbm,
           mean_hbm, out_hbm, uidx, iidx, urows, irows, ubv, ibv, meanv,
           outv, sem, bsem):
        wid = lax.axis_index("s") * NUM_CORES + lax.axis_index("c")
        pltpu.sync_copy(uid_hbm.at[wid], uidx)
        pltpu.sync_copy(iid_hbm.at[wid], iidx)
        pltpu.sync_copy(mean_hbm, meanv)
        bias_copies = []
        for c in range(n_chunks):
            sl = pl.ds(c * IDX_CHUNK, IDX_CHUNK)
            bias_copies.append(
                pltpu.async_copy(ubias_hbm.at[uidx.at[c]], ubv.at[sl], bsem))
            bias_copies.append(
                pltpu.async_copy(ibias_hbm.at[iidx.at[c]], ibv.at[sl], bsem))
        for cp in bias_copies:
            cp.wait()
        mean_vec = meanv[...]

        for c in range(n_chunks):
            copies = []
            for j in range(SIZE):
                copies.append(pltpu.async_copy(
                    uebd_hbm.at[j // 8, j % 8].at[uidx.at[c]],
                    urows.at[j], sem))
                copies.append(pltpu.async_copy(
                    iebd_hbm.at[j // 8, j % 8].at[iidx.at[c]],
                    irows.at[j], sem))
            for cp in copies:
                cp.wait()

            def cgroup(g, carry):
                sl = pl.ds(g * LANES, LANES)
                acc = jnp.zeros((LANES,), jnp.float32)
                for j in range(SIZE):
                    acc = acc + urows[j, sl] * irows[j, sl]
                obase = pl.ds(c * IDX_CHUNK + g * LANES, LANES)
                outv[obase] = acc + ubv[obase] + ibv[obase] + mean_vec
                return carry

            lax.fori_loop(0, IDX_CHUNK // LANES, cgroup, 0)

        pltpu.sync_copy(outv, out_hbm.at[pl.ds(wid * b_per_w, b_per_w)])

    return mf(uid, iid, uebd_t, iebd_t, ubias, ibias, mean16)


def kernel(user_id, item_id, user_ebds, item_ebds, user_bias, item_bias, mean):
    batch = user_id.shape[0]
    b_per_w = batch // NUM_WORKERS
    uid = user_id.astype(jnp.int32).reshape(
        NUM_WORKERS, b_per_w // IDX_CHUNK, IDX_CHUNK)
    iid = item_id.astype(jnp.int32).reshape(
        NUM_WORKERS, b_per_w // IDX_CHUNK, IDX_CHUNK)
    mean16 = jnp.broadcast_to(mean.astype(jnp.float32), (LANES,))
    uebd3 = user_ebds.T.reshape(4, 8, user_ebds.shape[0])
    iebd3 = item_ebds.T.reshape(4, 8, item_ebds.shape[0])
    return _mf_sc(batch, uid, iid, uebd3, iebd3,
                  user_bias.reshape(-1), item_bias.reshape(-1), mean16)


# speedup vs baseline: 3.5759x; 3.5705x over previous
import functools

import jax
import jax.numpy as jnp
from jax import lax
from jax.experimental import pallas as pl
from jax.experimental.pallas import tpu as pltpu
from jax.experimental.pallas import tpu_sc as plsc

SIZE = 32
LANES = 16
NUM_CORES = 2
NUM_SUBCORES = 16
NUM_WORKERS = NUM_CORES * NUM_SUBCORES
IDX_CHUNK = 128

VOCAB = 1000000
TILE_W = 128
FULL_TC = VOCAB // TILE_W
TAIL = VOCAB - FULL_TC * TILE_W
TC_PER_W = FULL_TC // NUM_WORKERS
TC_REM = FULL_TC - TC_PER_W * NUM_WORKERS
SLAB = 2048


@jax.jit
def _detile_sc(uebd_t, iebd_t, utail, itail):
    mesh = plsc.VectorSubcoreMesh(core_axis_name="c", subcore_axis_name="s")

    @functools.partial(
        pl.kernel,
        mesh=mesh,
        compiler_params=pltpu.CompilerParams(needs_layout_passes=False),
        out_type=(
            jax.ShapeDtypeStruct((SIZE * VOCAB,), jnp.float32),
            jax.ShapeDtypeStruct((SIZE * VOCAB,), jnp.float32),
        ),
        scratch_types=[
            pltpu.VMEM((2, 8, TILE_W), jnp.float32),
            pltpu.VMEM((SIZE * TAIL,), jnp.float32),
            pltpu.SemaphoreType.DMA,
            pltpu.SemaphoreType.DMA,
        ],
    )
    def detile(ut_hbm, it_hbm, utail_hbm, itail_hbm, us_hbm, is_hbm,
               vtile, vtail, insem, sem):
        wid = lax.axis_index("s") * NUM_CORES + lax.axis_index("c")
        base_tc = wid * TC_PER_W

        def emit(src_hbm, dst_hbm, tc):
            for tj in range(4):
                buf = vtile.at[tj % 2]
                pltpu.async_copy(
                    src_hbm.at[pl.ds(tj * 8, 8),
                               pl.ds(tc * TILE_W, TILE_W)],
                    buf, insem).wait()
                copies = []
                for r in range(8):
                    j = tj * 8 + r
                    copies.append(pltpu.async_copy(
                        buf.at[r],
                        dst_hbm.at[pl.ds(j * VOCAB + tc * TILE_W, TILE_W)],
                        sem))
                for cp in copies:
                    cp.wait()

        def tc_loop(k, carry):
            emit(ut_hbm, us_hbm, base_tc + k)
            emit(it_hbm, is_hbm, base_tc + k)
            return carry

        lax.fori_loop(0, TC_PER_W, tc_loop, 0)

        @pl.when(wid == NUM_WORKERS - 1)
        def _():
            for k in range(TC_REM):
                emit(ut_hbm, us_hbm, NUM_WORKERS * TC_PER_W + k)
                emit(it_hbm, is_hbm, NUM_WORKERS * TC_PER_W + k)
            for tail_hbm, dst_hbm in ((utail_hbm, us_hbm),
                                      (itail_hbm, is_hbm)):
                pltpu.sync_copy(tail_hbm, vtail)
                tcopies = []
                for j in range(SIZE):
                    tcopies.append(pltpu.async_copy(
                        vtail.at[pl.ds(j * TAIL, TAIL)],
                        dst_hbm.at[pl.ds(j * VOCAB + FULL_TC * TILE_W,
                                         TAIL)],
                        sem))
                for cp in tcopies:
                    cp.wait()

    return detile(uebd_t, iebd_t, utail, itail)


@functools.partial(jax.jit, static_argnums=0)
def _mf_sc(batch, uid, iid, us, is_, ubias, ibias, mean16):
    b_per_w = batch // NUM_WORKERS
    n_chunks = b_per_w // IDX_CHUNK
    mesh = plsc.VectorSubcoreMesh(core_axis_name="c", subcore_axis_name="s")

    @functools.partial(
        pl.kernel,
        mesh=mesh,
        compiler_params=pltpu.CompilerParams(
            needs_layout_passes=False, use_tc_tiling_on_sc=False),
        out_type=jax.ShapeDtypeStruct((batch,), jnp.float32),
        scratch_types=[
            pltpu.VMEM((n_chunks, IDX_CHUNK), jnp.int32),
            pltpu.VMEM((n_chunks, IDX_CHUNK), jnp.int32),
            pltpu.VMEM((SIZE, IDX_CHUNK), jnp.float32),
            pltpu.VMEM((SIZE, IDX_CHUNK), jnp.float32),
            pltpu.VMEM((b_per_w,), jnp.float32),
            pltpu.VMEM((b_per_w,), jnp.float32),
            pltpu.VMEM((LANES,), jnp.float32),
            pltpu.VMEM((b_per_w,), jnp.float32),
            pltpu.SemaphoreType.DMA,
            pltpu.SemaphoreType.DMA,
        ],
    )
    def mf(uid_hbm, iid_hbm, us_hbm, is_hbm, ubias_hbm, ibias_hbm,
           mean_hbm, out_hbm, uidx, iidx, urows, irows, ubv, ibv, meanv,
           outv, sem, bsem):
        wid = lax.axis_index("s") * NUM_CORES + lax.axis_index("c")
        pltpu.sync_copy(uid_hbm.at[wid], uidx)
        pltpu.sync_copy(iid_hbm.at[wid], iidx)
        pltpu.sync_copy(mean_hbm, meanv)
        bias_copies = []
        for c in range(n_chunks):
            sl = pl.ds(c * IDX_CHUNK, IDX_CHUNK)
            bias_copies.append(
                pltpu.async_copy(ubias_hbm.at[uidx.at[c]], ubv.at[sl], bsem))
            bias_copies.append(
                pltpu.async_copy(ibias_hbm.at[iidx.at[c]], ibv.at[sl], bsem))
        for cp in bias_copies:
            cp.wait()
        mean_vec = meanv[...]

        for c in range(n_chunks):
            copies = []
            for j in range(SIZE):
                copies.append(pltpu.async_copy(
                    us_hbm.at[pl.ds(j * VOCAB, VOCAB)].at[uidx.at[c]],
                    urows.at[j], sem))
                copies.append(pltpu.async_copy(
                    is_hbm.at[pl.ds(j * VOCAB, VOCAB)].at[iidx.at[c]],
                    irows.at[j], sem))
            for cp in copies:
                cp.wait()

            def cgroup(g, carry):
                sl = pl.ds(g * LANES, LANES)
                acc = jnp.zeros((LANES,), jnp.float32)
                for j in range(SIZE):
                    acc = acc + urows[j, sl] * irows[j, sl]
                obase = pl.ds(c * IDX_CHUNK + g * LANES, LANES)
                outv[obase] = acc + ubv[obase] + ibv[obase] + mean_vec
                return carry

            lax.fori_loop(0, IDX_CHUNK // LANES, cgroup, 0)

        pltpu.sync_copy(outv, out_hbm.at[pl.ds(wid * b_per_w, b_per_w)])

    return mf(uid, iid, us, is_, ubias, ibias, mean16)


def kernel(user_id, item_id, user_ebds, item_ebds, user_bias, item_bias, mean):
    batch = user_id.shape[0]
    b_per_w = batch // NUM_WORKERS
    uid = user_id.astype(jnp.int32).reshape(
        NUM_WORKERS, b_per_w // IDX_CHUNK, IDX_CHUNK)
    iid = item_id.astype(jnp.int32).reshape(
        NUM_WORKERS, b_per_w // IDX_CHUNK, IDX_CHUNK)
    mean16 = jnp.broadcast_to(mean.astype(jnp.float32), (LANES,))
    ut = user_ebds.T
    it = item_ebds.T
    utail = ut[:, FULL_TC * TILE_W:].reshape(-1)
    itail = it[:, FULL_TC * TILE_W:].reshape(-1)
    us, is_ = _detile_sc(ut, it, utail, itail)
    return _mf_sc(batch, uid, iid, us, is_,
                  user_bias.reshape(-1), item_bias.reshape(-1), mean16)


# speedup vs baseline: 11.5190x; 3.2213x over previous
import functools

import jax
import jax.numpy as jnp
from jax import lax
from jax.experimental import pallas as pl
from jax.experimental.pallas import tpu as pltpu
from jax.experimental.pallas import tpu_sc as plsc

SIZE = 32
LANES = 16
NUM_CORES = 2
NUM_SUBCORES = 16
NUM_WORKERS = NUM_CORES * NUM_SUBCORES
IDX_CHUNK = 128

VOCAB = 1000000
TILE_W = 128
FULL_TC = VOCAB // TILE_W
TAIL = VOCAB - FULL_TC * TILE_W
TC_PER_W = FULL_TC // NUM_WORKERS
TC_REM = FULL_TC - TC_PER_W * NUM_WORKERS
SLAB = 2048


@jax.jit
def _detile_sc(uebd_t, iebd_t, utail, itail):
    mesh = plsc.VectorSubcoreMesh(core_axis_name="c", subcore_axis_name="s")

    @functools.partial(
        pl.kernel,
        mesh=mesh,
        compiler_params=pltpu.CompilerParams(needs_layout_passes=False),
        out_type=(
            jax.ShapeDtypeStruct((SIZE * VOCAB,), jnp.float32),
            jax.ShapeDtypeStruct((SIZE * VOCAB,), jnp.float32),
        ),
        scratch_types=[
            pltpu.VMEM((2, 4, 8, TILE_W), jnp.float32),
            pltpu.VMEM((SIZE * TAIL,), jnp.float32),
            pltpu.SemaphoreType.DMA,
            pltpu.SemaphoreType.DMA,
        ],
    )
    def detile(ut_hbm, it_hbm, utail_hbm, itail_hbm, us_hbm, is_hbm,
               vtile, vtail, insem, sem):
        wid = lax.axis_index("s") * NUM_CORES + lax.axis_index("c")
        base_tc = wid * TC_PER_W

        def fire_in(src_hbm, tc, half):
            for tj in range(4):
                pltpu.async_copy(
                    src_hbm.at[pl.ds(tj * 8, 8),
                               pl.ds(tc * TILE_W, TILE_W)],
                    vtile.at[half, tj], insem)

        def wait_in():
            for tj in range(4):
                pltpu.make_async_copy(
                    ut_hbm.at[pl.ds(0, 8), pl.ds(0, TILE_W)],
                    vtile.at[0, 0], insem).wait()

        def fire_out(dst_hbm, tc, half):
            for j in range(SIZE):
                pltpu.async_copy(
                    vtile.at[half, j // 8, j % 8],
                    dst_hbm.at[pl.ds(j * VOCAB + tc * TILE_W, TILE_W)],
                    sem)

        def wait_out():
            for j in range(SIZE):
                pltpu.make_async_copy(
                    vtile.at[0, 0, 0],
                    us_hbm.at[pl.ds(0, TILE_W)], sem).wait()

        def tc_loop(k, carry):
            tc = base_tc + k
            fire_in(ut_hbm, tc, 0)
            fire_in(it_hbm, tc, 1)
            wait_in()
            wait_in()
            fire_out(us_hbm, tc, 0)
            fire_out(is_hbm, tc, 1)
            wait_out()
            wait_out()
            return carry

        lax.fori_loop(0, TC_PER_W, tc_loop, 0)

        @pl.when(wid == NUM_WORKERS - 1)
        def _():
            for k in range(TC_REM):
                tc_loop(NUM_WORKERS * TC_PER_W + k - base_tc, 0)
            for tail_hbm, dst_hbm in ((utail_hbm, us_hbm),
                                      (itail_hbm, is_hbm)):
                pltpu.sync_copy(tail_hbm, vtail)
                tcopies = []
                for j in range(SIZE):
                    tcopies.append(pltpu.async_copy(
                        vtail.at[pl.ds(j * TAIL, TAIL)],
                        dst_hbm.at[pl.ds(j * VOCAB + FULL_TC * TILE_W,
                                         TAIL)],
                        sem))
                for cp in tcopies:
                    cp.wait()

    return detile(uebd_t, iebd_t, utail, itail)


@functools.partial(jax.jit, static_argnums=0)
def _mf_sc(batch, uid, iid, us, is_, ubias, ibias, mean16):
    b_per_w = batch // NUM_WORKERS
    n_chunks = b_per_w // IDX_CHUNK
    mesh = plsc.VectorSubcoreMesh(core_axis_name="c", subcore_axis_name="s")

    @functools.partial(
        pl.kernel,
        mesh=mesh,
        compiler_params=pltpu.CompilerParams(
            needs_layout_passes=False, use_tc_tiling_on_sc=False),
        out_type=jax.ShapeDtypeStruct((batch,), jnp.float32),
        scratch_types=[
            pltpu.VMEM((n_chunks, IDX_CHUNK), jnp.int32),
            pltpu.VMEM((n_chunks, IDX_CHUNK), jnp.int32),
            pltpu.VMEM((SIZE, IDX_CHUNK), jnp.float32),
            pltpu.VMEM((SIZE, IDX_CHUNK), jnp.float32),
            pltpu.VMEM((b_per_w,), jnp.float32),
            pltpu.VMEM((b_per_w,), jnp.float32),
            pltpu.VMEM((LANES,), jnp.float32),
            pltpu.VMEM((b_per_w,), jnp.float32),
            pltpu.SemaphoreType.DMA,
            pltpu.SemaphoreType.DMA,
        ],
    )
    def mf(uid_hbm, iid_hbm, us_hbm, is_hbm, ubias_hbm, ibias_hbm,
           mean_hbm, out_hbm, uidx, iidx, urows, irows, ubv, ibv, meanv,
           outv, sem, bsem):
        wid = lax.axis_index("s") * NUM_CORES + lax.axis_index("c")
        pltpu.sync_copy(uid_hbm.at[wid], uidx)
        pltpu.sync_copy(iid_hbm.at[wid], iidx)
        pltpu.sync_copy(mean_hbm, meanv)
        bias_copies = []
        for c in range(n_chunks):
            sl = pl.ds(c * IDX_CHUNK, IDX_CHUNK)
            bias_copies.append(
                pltpu.async_copy(ubias_hbm.at[uidx.at[c]], ubv.at[sl], bsem))
            bias_copies.append(
                pltpu.async_copy(ibias_hbm.at[iidx.at[c]], ibv.at[sl], bsem))
        for cp in bias_copies:
            cp.wait()
        mean_vec = meanv[...]

        for c in range(n_chunks):
            copies = []
            for j in range(SIZE):
                copies.append(pltpu.async_copy(
                    us_hbm.at[pl.ds(j * VOCAB, VOCAB)].at[uidx.at[c]],
                    urows.at[j], sem))
                copies.append(pltpu.async_copy(
                    is_hbm.at[pl.ds(j * VOCAB, VOCAB)].at[iidx.at[c]],
                    irows.at[j], sem))
            for cp in copies:
                cp.wait()

            def cgroup(g, carry):
                sl = pl.ds(g * LANES, LANES)
                acc = jnp.zeros((LANES,), jnp.float32)
                for j in range(SIZE):
                    acc = acc + urows[j, sl] * irows[j, sl]
                obase = pl.ds(c * IDX_CHUNK + g * LANES, LANES)
                outv[obase] = acc + ubv[obase] + ibv[obase] + mean_vec
                return carry

            lax.fori_loop(0, IDX_CHUNK // LANES, cgroup, 0)

        pltpu.sync_copy(outv, out_hbm.at[pl.ds(wid * b_per_w, b_per_w)])

    return mf(uid, iid, us, is_, ubias, ibias, mean16)


def kernel(user_id, item_id, user_ebds, item_ebds, user_bias, item_bias, mean):
    batch = user_id.shape[0]
    b_per_w = batch // NUM_WORKERS
    uid = user_id.astype(jnp.int32).reshape(
        NUM_WORKERS, b_per_w // IDX_CHUNK, IDX_CHUNK)
    iid = item_id.astype(jnp.int32).reshape(
        NUM_WORKERS, b_per_w // IDX_CHUNK, IDX_CHUNK)
    mean16 = jnp.broadcast_to(mean.astype(jnp.float32), (LANES,))
    ut = user_ebds.T
    it = item_ebds.T
    utail = ut[:, FULL_TC * TILE_W:].reshape(-1)
    itail = it[:, FULL_TC * TILE_W:].reshape(-1)
    us, is_ = _detile_sc(ut, it, utail, itail)
    return _mf_sc(batch, uid, iid, us, is_,
                  user_bias.reshape(-1), item_bias.reshape(-1), mean16)


# speedup vs baseline: 17.3838x; 1.5091x over previous
import functools

import jax
import jax.numpy as jnp
from jax import lax
from jax.experimental import pallas as pl
from jax.experimental.pallas import tpu as pltpu
from jax.experimental.pallas import tpu_sc as plsc

SIZE = 32
LANES = 16
NUM_CORES = 2
NUM_SUBCORES = 16
NUM_WORKERS = NUM_CORES * NUM_SUBCORES
IDX_CHUNK = 128

VOCAB = 1000000
TILE_W = 128
FULL_TC = VOCAB // TILE_W
TAIL = VOCAB - FULL_TC * TILE_W
TC_PER_W = FULL_TC // NUM_WORKERS
TC_REM = FULL_TC - TC_PER_W * NUM_WORKERS
SLAB = 2048


@jax.jit
def _detile_sc(uebd_t, iebd_t, utail, itail):
    mesh = plsc.VectorSubcoreMesh(core_axis_name="c", subcore_axis_name="s")

    @functools.partial(
        pl.kernel,
        mesh=mesh,
        compiler_params=pltpu.CompilerParams(needs_layout_passes=False),
        out_type=(
            jax.ShapeDtypeStruct((SIZE * VOCAB,), jnp.float32),
            jax.ShapeDtypeStruct((SIZE * VOCAB,), jnp.float32),
        ),
        scratch_types=[
            pltpu.VMEM((2, 2, 4, 8, TILE_W), jnp.float32),
            pltpu.VMEM((SIZE * TAIL,), jnp.float32),
            pltpu.SemaphoreType.DMA,
            pltpu.SemaphoreType.DMA,
        ],
    )
    def detile(ut_hbm, it_hbm, utail_hbm, itail_hbm, us_hbm, is_hbm,
               vtile, vtail, insem, sem):
        wid = lax.axis_index("s") * NUM_CORES + lax.axis_index("c")
        base_tc = wid * TC_PER_W

        def fire_in(tc, ph):
            for t, src_hbm in enumerate((ut_hbm, it_hbm)):
                for tj in range(4):
                    pltpu.async_copy(
                        src_hbm.at[pl.ds(tj * 8, 8),
                                   pl.ds(tc * TILE_W, TILE_W)],
                        vtile.at[ph, t, tj], insem)

        def wait_in():
            for _ in range(8):
                pltpu.make_async_copy(
                    ut_hbm.at[pl.ds(0, 8), pl.ds(0, TILE_W)],
                    vtile.at[0, 0, 0], insem).wait()

        def fire_out(tc, ph):
            for t, dst_hbm in enumerate((us_hbm, is_hbm)):
                for j in range(SIZE):
                    pltpu.async_copy(
                        vtile.at[ph, t, j // 8, j % 8],
                        dst_hbm.at[pl.ds(j * VOCAB + tc * TILE_W, TILE_W)],
                        sem)

        def wait_out():
            for _ in range(2 * SIZE):
                pltpu.make_async_copy(
                    vtile.at[0, 0, 0, 0],
                    us_hbm.at[pl.ds(0, TILE_W)], sem).wait()

        fire_in(base_tc, 0)

        def tc_loop2(k, carry):
            tc = base_tc + k
            ph = lax.rem(k, 2)

            @pl.when(k > 0)
            def _():
                wait_out()
            fire_in(tc + 1, 1 - ph)
            wait_in()
            fire_out(tc, ph)
            return carry

        lax.fori_loop(0, TC_PER_W, tc_loop2, 0)
        wait_out()
        wait_in()

        @pl.when(wid == NUM_WORKERS - 1)
        def _():
            for k in range(TC_REM):
                tc = NUM_WORKERS * TC_PER_W + k
                fire_in(tc, 0)
                wait_in()
                fire_out(tc, 0)
                wait_out()
            for tail_hbm, dst_hbm in ((utail_hbm, us_hbm),
                                      (itail_hbm, is_hbm)):
                pltpu.sync_copy(tail_hbm, vtail)
                tcopies = []
                for j in range(SIZE):
                    tcopies.append(pltpu.async_copy(
                        vtail.at[pl.ds(j * TAIL, TAIL)],
                        dst_hbm.at[pl.ds(j * VOCAB + FULL_TC * TILE_W,
                                         TAIL)],
                        sem))
                for cp in tcopies:
                    cp.wait()

    return detile(uebd_t, iebd_t, utail, itail)


@functools.partial(jax.jit, static_argnums=0)
def _mf_sc(batch, uid, iid, us, is_, ubias, ibias, mean16):
    b_per_w = batch // NUM_WORKERS
    n_chunks = b_per_w // IDX_CHUNK
    mesh = plsc.VectorSubcoreMesh(core_axis_name="c", subcore_axis_name="s")

    @functools.partial(
        pl.kernel,
        mesh=mesh,
        compiler_params=pltpu.CompilerParams(
            needs_layout_passes=False, use_tc_tiling_on_sc=False),
        out_type=jax.ShapeDtypeStruct((batch,), jnp.float32),
        scratch_types=[
            pltpu.VMEM((n_chunks, IDX_CHUNK), jnp.int32),
            pltpu.VMEM((n_chunks, IDX_CHUNK), jnp.int32),
            pltpu.VMEM((SIZE, IDX_CHUNK), jnp.float32),
            pltpu.VMEM((SIZE, IDX_CHUNK), jnp.float32),
            pltpu.VMEM((b_per_w,), jnp.float32),
            pltpu.VMEM((b_per_w,), jnp.float32),
            pltpu.VMEM((LANES,), jnp.float32),
            pltpu.VMEM((b_per_w,), jnp.float32),
            pltpu.SemaphoreType.DMA,
            pltpu.SemaphoreType.DMA,
        ],
    )
    def mf(uid_hbm, iid_hbm, us_hbm, is_hbm, ubias_hbm, ibias_hbm,
           mean_hbm, out_hbm, uidx, iidx, urows, irows, ubv, ibv, meanv,
           outv, sem, bsem):
        wid = lax.axis_index("s") * NUM_CORES + lax.axis_index("c")
        pltpu.sync_copy(uid_hbm.at[wid], uidx)
        pltpu.sync_copy(iid_hbm.at[wid], iidx)
        pltpu.sync_copy(mean_hbm, meanv)
        bias_copies = []
        for c in range(n_chunks):
            sl = pl.ds(c * IDX_CHUNK, IDX_CHUNK)
            bias_copies.append(
                pltpu.async_copy(ubias_hbm.at[uidx.at[c]], ubv.at[sl], bsem))
            bias_copies.append(
                pltpu.async_copy(ibias_hbm.at[iidx.at[c]], ibv.at[sl], bsem))
        for cp in bias_copies:
            cp.wait()
        mean_vec = meanv[...]

        for c in range(n_chunks):
            copies = []
            for j in range(SIZE):
                copies.append(pltpu.async_copy(
                    us_hbm.at[pl.ds(j * VOCAB, VOCAB)].at[uidx.at[c]],
                    urows.at[j], sem))
                copies.append(pltpu.async_copy(
                    is_hbm.at[pl.ds(j * VOCAB, VOCAB)].at[iidx.at[c]],
                    irows.at[j], sem))
            for cp in copies:
                cp.wait()

            def cgroup(g, carry):
                sl = pl.ds(g * LANES, LANES)
                acc = jnp.zeros((LANES,), jnp.float32)
                for j in range(SIZE):
                    acc = acc + urows[j, sl] * irows[j, sl]
                obase = pl.ds(c * IDX_CHUNK + g * LANES, LANES)
                outv[obase] = acc + ubv[obase] + ibv[obase] + mean_vec
                return carry

            lax.fori_loop(0, IDX_CHUNK // LANES, cgroup, 0)

        pltpu.sync_copy(outv, out_hbm.at[pl.ds(wid * b_per_w, b_per_w)])

    return mf(uid, iid, us, is_, ubias, ibias, mean16)


def kernel(user_id, item_id, user_ebds, item_ebds, user_bias, item_bias, mean):
    batch = user_id.shape[0]
    b_per_w = batch // NUM_WORKERS
    uid = user_id.astype(jnp.int32).reshape(
        NUM_WORKERS, b_per_w // IDX_CHUNK, IDX_CHUNK)
    iid = item_id.astype(jnp.int32).reshape(
        NUM_WORKERS, b_per_w // IDX_CHUNK, IDX_CHUNK)
    mean16 = jnp.broadcast_to(mean.astype(jnp.float32), (LANES,))
    ut = user_ebds.T
    it = item_ebds.T
    utail = ut[:, FULL_TC * TILE_W:].reshape(-1)
    itail = it[:, FULL_TC * TILE_W:].reshape(-1)
    us, is_ = _detile_sc(ut, it, utail, itail)
    return _mf_sc(batch, uid, iid, us, is_,
                  user_bias.reshape(-1), item_bias.reshape(-1), mean16)


# speedup vs baseline: 17.4541x; 1.0040x over previous
import functools

import jax
import jax.numpy as jnp
from jax import lax
from jax.experimental import pallas as pl
from jax.experimental.pallas import tpu as pltpu
from jax.experimental.pallas import tpu_sc as plsc

SIZE = 32
LANES = 16
NUM_CORES = 2
NUM_SUBCORES = 16
NUM_WORKERS = NUM_CORES * NUM_SUBCORES
IDX_CHUNK = 128

VOCAB = 1000000
TILE_W = 128
FULL_TC = VOCAB // TILE_W
TAIL = VOCAB - FULL_TC * TILE_W
TC_PER_W = FULL_TC // NUM_WORKERS
TC_REM = FULL_TC - TC_PER_W * NUM_WORKERS
SLAB = 2048


@jax.jit
def _detile_sc(uebd_t, iebd_t, utail, itail):
    mesh = plsc.VectorSubcoreMesh(core_axis_name="c", subcore_axis_name="s")

    @functools.partial(
        pl.kernel,
        mesh=mesh,
        compiler_params=pltpu.CompilerParams(needs_layout_passes=False),
        out_type=(
            jax.ShapeDtypeStruct((SIZE * VOCAB,), jnp.float32),
            jax.ShapeDtypeStruct((SIZE * VOCAB,), jnp.float32),
        ),
        scratch_types=[
            pltpu.VMEM((2, 2, 4, 8, TILE_W), jnp.float32),
            pltpu.VMEM((SIZE * TAIL,), jnp.float32),
            pltpu.SemaphoreType.DMA,
            pltpu.SemaphoreType.DMA,
        ],
    )
    def detile(ut_hbm, it_hbm, utail_hbm, itail_hbm, us_hbm, is_hbm,
               vtile, vtail, insem, sem):
        wid = lax.axis_index("s") * NUM_CORES + lax.axis_index("c")
        base_tc = wid * TC_PER_W

        def fire_in(tc, ph):
            for t, src_hbm in enumerate((ut_hbm, it_hbm)):
                for tj in range(4):
                    pltpu.async_copy(
                        src_hbm.at[pl.ds(tj * 8, 8),
                                   pl.ds(tc * TILE_W, TILE_W)],
                        vtile.at[ph, t, tj], insem)

        def wait_in():
            for _ in range(8):
                pltpu.make_async_copy(
                    ut_hbm.at[pl.ds(0, 8), pl.ds(0, TILE_W)],
                    vtile.at[0, 0, 0], insem).wait()

        def fire_out(tc, ph):
            for t, dst_hbm in enumerate((us_hbm, is_hbm)):
                for j in range(SIZE):
                    pltpu.async_copy(
                        vtile.at[ph, t, j // 8, j % 8],
                        dst_hbm.at[pl.ds(j * VOCAB + tc * TILE_W, TILE_W)],
                        sem)

        def wait_out():
            for _ in range(2 * SIZE):
                pltpu.make_async_copy(
                    vtile.at[0, 0, 0, 0],
                    us_hbm.at[pl.ds(0, TILE_W)], sem).wait()

        fire_in(base_tc, 0)

        def tc_loop2(k, carry):
            tc = base_tc + k
            ph = lax.rem(k, 2)

            @pl.when(k > 0)
            def _():
                wait_out()
            fire_in(tc + 1, 1 - ph)
            wait_in()
            fire_out(tc, ph)
            return carry

        lax.fori_loop(0, TC_PER_W, tc_loop2, 0)
        wait_out()
        wait_in()

        @pl.when(wid == NUM_WORKERS - 1)
        def _():
            for k in range(TC_REM):
                tc = NUM_WORKERS * TC_PER_W + k
                fire_in(tc, 0)
                wait_in()
                fire_out(tc, 0)
                wait_out()
            for tail_hbm, dst_hbm in ((utail_hbm, us_hbm),
                                      (itail_hbm, is_hbm)):
                pltpu.sync_copy(tail_hbm, vtail)
                tcopies = []
                for j in range(SIZE):
                    tcopies.append(pltpu.async_copy(
                        vtail.at[pl.ds(j * TAIL, TAIL)],
                        dst_hbm.at[pl.ds(j * VOCAB + FULL_TC * TILE_W,
                                         TAIL)],
                        sem))
                for cp in tcopies:
                    cp.wait()

    return detile(uebd_t, iebd_t, utail, itail)


@functools.partial(jax.jit, static_argnums=0)
def _mf_sc(batch, uid, iid, us, is_, ubias, ibias, mean16):
    b_per_w = batch // NUM_WORKERS
    n_chunks = b_per_w // IDX_CHUNK
    mesh = plsc.VectorSubcoreMesh(core_axis_name="c", subcore_axis_name="s")

    @functools.partial(
        pl.kernel,
        mesh=mesh,
        compiler_params=pltpu.CompilerParams(
            needs_layout_passes=False, use_tc_tiling_on_sc=False),
        out_type=jax.ShapeDtypeStruct((batch,), jnp.float32),
        scratch_types=[
            pltpu.VMEM((n_chunks, IDX_CHUNK), jnp.int32),
            pltpu.VMEM((n_chunks, IDX_CHUNK), jnp.int32),
            pltpu.VMEM((2, SIZE, IDX_CHUNK), jnp.float32),
            pltpu.VMEM((2, SIZE, IDX_CHUNK), jnp.float32),
            pltpu.VMEM((b_per_w,), jnp.float32),
            pltpu.VMEM((b_per_w,), jnp.float32),
            pltpu.VMEM((LANES,), jnp.float32),
            pltpu.VMEM((b_per_w,), jnp.float32),
            pltpu.SemaphoreType.DMA,
            pltpu.SemaphoreType.DMA,
        ],
    )
    def mf(uid_hbm, iid_hbm, us_hbm, is_hbm, ubias_hbm, ibias_hbm,
           mean_hbm, out_hbm, uidx, iidx, urows, irows, ubv, ibv, meanv,
           outv, sem, bsem):
        wid = lax.axis_index("s") * NUM_CORES + lax.axis_index("c")
        pltpu.sync_copy(uid_hbm.at[wid], uidx)
        pltpu.sync_copy(iid_hbm.at[wid], iidx)
        pltpu.sync_copy(mean_hbm, meanv)
        bias_copies = []
        for c in range(n_chunks):
            sl = pl.ds(c * IDX_CHUNK, IDX_CHUNK)
            bias_copies.append(
                pltpu.async_copy(ubias_hbm.at[uidx.at[c]], ubv.at[sl], bsem))
            bias_copies.append(
                pltpu.async_copy(ibias_hbm.at[iidx.at[c]], ibv.at[sl], bsem))
        for cp in bias_copies:
            cp.wait()
        mean_vec = meanv[...]

        def fire_gathers(c, ph):
            for j in range(SIZE):
                pltpu.async_copy(
                    us_hbm.at[pl.ds(j * VOCAB, VOCAB)].at[uidx.at[c]],
                    urows.at[ph, j], sem)
                pltpu.async_copy(
                    is_hbm.at[pl.ds(j * VOCAB, VOCAB)].at[iidx.at[c]],
                    irows.at[ph, j], sem)

        def wait_gathers():
            for _ in range(2 * SIZE):
                pltpu.make_async_copy(
                    us_hbm.at[pl.ds(0, IDX_CHUNK)],
                    urows.at[0, 0], sem).wait()

        fire_gathers(0, 0)
        for c in range(n_chunks):
            wait_gathers()
            if c + 1 < n_chunks:
                fire_gathers(c + 1, (c + 1) % 2)
            ph = c % 2

            def cgroup(g, carry):
                sl = pl.ds(g * LANES, LANES)
                acc = jnp.zeros((LANES,), jnp.float32)
                for j in range(SIZE):
                    acc = acc + urows[ph, j, sl] * irows[ph, j, sl]
                obase = pl.ds(c * IDX_CHUNK + g * LANES, LANES)
                outv[obase] = acc + ubv[obase] + ibv[obase] + mean_vec
                return carry

            lax.fori_loop(0, IDX_CHUNK // LANES, cgroup, 0)

        pltpu.sync_copy(outv, out_hbm.at[pl.ds(wid * b_per_w, b_per_w)])

    return mf(uid, iid, us, is_, ubias, ibias, mean16)


def kernel(user_id, item_id, user_ebds, item_ebds, user_bias, item_bias, mean):
    batch = user_id.shape[0]
    b_per_w = batch // NUM_WORKERS
    uid = user_id.astype(jnp.int32).reshape(
        NUM_WORKERS, b_per_w // IDX_CHUNK, IDX_CHUNK)
    iid = item_id.astype(jnp.int32).reshape(
        NUM_WORKERS, b_per_w // IDX_CHUNK, IDX_CHUNK)
    mean16 = jnp.broadcast_to(mean.astype(jnp.float32), (LANES,))
    ut = user_ebds.T
    it = item_ebds.T
    utail = ut[:, FULL_TC * TILE_W:].reshape(-1)
    itail = it[:, FULL_TC * TILE_W:].reshape(-1)
    us, is_ = _detile_sc(ut, it, utail, itail)
    return _mf_sc(batch, uid, iid, us, is_,
                  user_bias.reshape(-1), item_bias.reshape(-1), mean16)


# speedup vs baseline: 17.4884x; 1.0020x over previous
import functools

import jax
import jax.numpy as jnp
from jax import lax
from jax.experimental import pallas as pl
from jax.experimental.pallas import tpu as pltpu
from jax.experimental.pallas import tpu_sc as plsc

SIZE = 32
LANES = 16
NUM_CORES = 2
NUM_SUBCORES = 16
NUM_WORKERS = NUM_CORES * NUM_SUBCORES
IDX_CHUNK = 128

VOCAB = 1000000
TILE_W = 128
FULL_TC = VOCAB // TILE_W
TAIL = VOCAB - FULL_TC * TILE_W
TC_PER_W = FULL_TC // NUM_WORKERS
TC_REM = FULL_TC - TC_PER_W * NUM_WORKERS
SLAB = 2048


@jax.jit
def _detile_sc(uebd_t, iebd_t, utail, itail):
    mesh = plsc.VectorSubcoreMesh(core_axis_name="c", subcore_axis_name="s")

    @functools.partial(
        pl.kernel,
        mesh=mesh,
        compiler_params=pltpu.CompilerParams(needs_layout_passes=False),
        out_type=(
            jax.ShapeDtypeStruct((SIZE * VOCAB,), jnp.float32),
            jax.ShapeDtypeStruct((SIZE * VOCAB,), jnp.float32),
        ),
        scratch_types=[
            pltpu.VMEM((3, 2, 4, 8, TILE_W), jnp.float32),
            pltpu.VMEM((SIZE * TAIL,), jnp.float32),
            pltpu.SemaphoreType.DMA,
            pltpu.SemaphoreType.DMA,
        ],
    )
    def detile(ut_hbm, it_hbm, utail_hbm, itail_hbm, us_hbm, is_hbm,
               vtile, vtail, insem, sem):
        wid = lax.axis_index("s") * NUM_CORES + lax.axis_index("c")
        base_tc = wid * TC_PER_W

        def fire_in(tc, ph):
            for t, src_hbm in enumerate((ut_hbm, it_hbm)):
                for tj in range(4):
                    pltpu.async_copy(
                        src_hbm.at[pl.ds(tj * 8, 8),
                                   pl.ds(tc * TILE_W, TILE_W)],
                        vtile.at[ph, t, tj], insem)

        def wait_in():
            for _ in range(8):
                pltpu.make_async_copy(
                    ut_hbm.at[pl.ds(0, 8), pl.ds(0, TILE_W)],
                    vtile.at[0, 0, 0], insem).wait()

        def fire_out(tc, ph):
            for t, dst_hbm in enumerate((us_hbm, is_hbm)):
                for j in range(SIZE):
                    pltpu.async_copy(
                        vtile.at[ph, t, j // 8, j % 8],
                        dst_hbm.at[pl.ds(j * VOCAB + tc * TILE_W, TILE_W)],
                        sem)

        def wait_out():
            for _ in range(2 * SIZE):
                pltpu.make_async_copy(
                    vtile.at[0, 0, 0, 0],
                    us_hbm.at[pl.ds(0, TILE_W)], sem).wait()

        fire_in(base_tc, 0)

        def tc_loop2(k, carry):
            tc = base_tc + k
            ph = lax.rem(k, 3)
            fire_in(tc + 1, lax.rem(k + 1, 3))
            wait_in()
            fire_out(tc, ph)

            @pl.when(k > 0)
            def _():
                wait_out()
            return carry

        lax.fori_loop(0, TC_PER_W, tc_loop2, 0)
        wait_out()
        wait_in()

        @pl.when(wid == NUM_WORKERS - 1)
        def _():
            for k in range(TC_REM):
                tc = NUM_WORKERS * TC_PER_W + k
                fire_in(tc, 0)
                wait_in()
                fire_out(tc, 0)
                wait_out()
            for tail_hbm, dst_hbm in ((utail_hbm, us_hbm),
                                      (itail_hbm, is_hbm)):
                pltpu.sync_copy(tail_hbm, vtail)
                tcopies = []
                for j in range(SIZE):
                    tcopies.append(pltpu.async_copy(
                        vtail.at[pl.ds(j * TAIL, TAIL)],
                        dst_hbm.at[pl.ds(j * VOCAB + FULL_TC * TILE_W,
                                         TAIL)],
                        sem))
                for cp in tcopies:
                    cp.wait()

    return detile(uebd_t, iebd_t, utail, itail)


@functools.partial(jax.jit, static_argnums=0)
def _mf_sc(batch, uid, iid, us, is_, ubias, ibias, mean16):
    b_per_w = batch // NUM_WORKERS
    n_chunks = b_per_w // IDX_CHUNK
    mesh = plsc.VectorSubcoreMesh(core_axis_name="c", subcore_axis_name="s")

    @functools.partial(
        pl.kernel,
        mesh=mesh,
        compiler_params=pltpu.CompilerParams(
            needs_layout_passes=False, use_tc_tiling_on_sc=False),
        out_type=jax.ShapeDtypeStruct((batch,), jnp.float32),
        scratch_types=[
            pltpu.VMEM((n_chunks, IDX_CHUNK), jnp.int32),
            pltpu.VMEM((n_chunks, IDX_CHUNK), jnp.int32),
            pltpu.VMEM((2, SIZE, IDX_CHUNK), jnp.float32),
            pltpu.VMEM((2, SIZE, IDX_CHUNK), jnp.float32),
            pltpu.VMEM((b_per_w,), jnp.float32),
            pltpu.VMEM((b_per_w,), jnp.float32),
            pltpu.VMEM((LANES,), jnp.float32),
            pltpu.VMEM((b_per_w,), jnp.float32),
            pltpu.SemaphoreType.DMA,
            pltpu.SemaphoreType.DMA,
        ],
    )
    def mf(uid_hbm, iid_hbm, us_hbm, is_hbm, ubias_hbm, ibias_hbm,
           mean_hbm, out_hbm, uidx, iidx, urows, irows, ubv, ibv, meanv,
           outv, sem, bsem):
        wid = lax.axis_index("s") * NUM_CORES + lax.axis_index("c")
        pltpu.sync_copy(uid_hbm.at[wid], uidx)
        pltpu.sync_copy(iid_hbm.at[wid], iidx)
        pltpu.sync_copy(mean_hbm, meanv)
        bias_copies = []
        for c in range(n_chunks):
            sl = pl.ds(c * IDX_CHUNK, IDX_CHUNK)
            bias_copies.append(
                pltpu.async_copy(ubias_hbm.at[uidx.at[c]], ubv.at[sl], bsem))
            bias_copies.append(
                pltpu.async_copy(ibias_hbm.at[iidx.at[c]], ibv.at[sl], bsem))
        for cp in bias_copies:
            cp.wait()
        mean_vec = meanv[...]

        def fire_gathers(c, ph):
            for j in range(SIZE):
                pltpu.async_copy(
                    us_hbm.at[pl.ds(j * VOCAB, VOCAB)].at[uidx.at[c]],
                    urows.at[ph, j], sem)
                pltpu.async_copy(
                    is_hbm.at[pl.ds(j * VOCAB, VOCAB)].at[iidx.at[c]],
                    irows.at[ph, j], sem)

        def wait_gathers():
            for _ in range(2 * SIZE):
                pltpu.make_async_copy(
                    us_hbm.at[pl.ds(0, IDX_CHUNK)],
                    urows.at[0, 0], sem).wait()

        fire_gathers(0, 0)
        for c in range(n_chunks):
            wait_gathers()
            if c + 1 < n_chunks:
                fire_gathers(c + 1, (c + 1) % 2)
            ph = c % 2

            def cgroup(g, carry):
                sl = pl.ds(g * LANES, LANES)
                acc = jnp.zeros((LANES,), jnp.float32)
                for j in range(SIZE):
                    acc = acc + urows[ph, j, sl] * irows[ph, j, sl]
                obase = pl.ds(c * IDX_CHUNK + g * LANES, LANES)
                outv[obase] = acc + ubv[obase] + ibv[obase] + mean_vec
                return carry

            lax.fori_loop(0, IDX_CHUNK // LANES, cgroup, 0)

        pltpu.sync_copy(outv, out_hbm.at[pl.ds(wid * b_per_w, b_per_w)])

    return mf(uid, iid, us, is_, ubias, ibias, mean16)


def kernel(user_id, item_id, user_ebds, item_ebds, user_bias, item_bias, mean):
    batch = user_id.shape[0]
    b_per_w = batch // NUM_WORKERS
    uid = user_id.astype(jnp.int32).reshape(
        NUM_WORKERS, b_per_w // IDX_CHUNK, IDX_CHUNK)
    iid = item_id.astype(jnp.int32).reshape(
        NUM_WORKERS, b_per_w // IDX_CHUNK, IDX_CHUNK)
    mean16 = jnp.broadcast_to(mean.astype(jnp.float32), (LANES,))
    ut = user_ebds.T
    it = item_ebds.T
    utail = ut[:, FULL_TC * TILE_W:].reshape(-1)
    itail = it[:, FULL_TC * TILE_W:].reshape(-1)
    us, is_ = _detile_sc(ut, it, utail, itail)
    return _mf_sc(batch, uid, iid, us, is_,
                  user_bias.reshape(-1), item_bias.reshape(-1), mean16)
